# baseline (speedup 1.0000x reference)
import functools

import jax
import jax.numpy as jnp
from jax import lax
from jax.experimental import pallas as pl
from jax.experimental.pallas import tpu as pltpu
from jax.experimental.pallas import tpu_sc as plsc

B = 16384
D = 64
L = 16
P = 2 * D

_info = plsc.get_sparse_core_info()
NC = _info.num_cores
NS = _info.num_subcores
NW = NC * NS
BPW = B // NW
CH = 128
NCH = BPW // CH
NG = CH // L

_mesh = plsc.VectorSubcoreMesh(core_axis_name="c", subcore_axis_name="s")


@functools.partial(
    pl.kernel,
    mesh=_mesh,
    out_type=jax.ShapeDtypeStruct((B,), jnp.float32),
    compiler_params=pltpu.CompilerParams(
        needs_layout_passes=False, use_tc_tiling_on_sc=True),
    scratch_types=[
        pltpu.VMEM((BPW,), jnp.int32),
        pltpu.VMEM((BPW,), jnp.int32),
        pltpu.VMEM((BPW,), jnp.int32),
        pltpu.VMEM((NCH, CH), jnp.int32),
        pltpu.VMEM((NCH, CH), jnp.int32),
        pltpu.VMEM((NCH, CH), jnp.int32),
        pltpu.VMEM((CH, P), jnp.float32),
        pltpu.VMEM((CH, P), jnp.float32),
        pltpu.VMEM((CH, P), jnp.float32),
        pltpu.VMEM((CH, P), jnp.float32),
        pltpu.VMEM((CH, P), jnp.float32),
        pltpu.VMEM((CH, P), jnp.float32),
        pltpu.VMEM((BPW,), jnp.float32),
        pltpu.SemaphoreType.DMA,
        pltpu.SemaphoreType.DMA,
    ],
)
def _transe_sc(heads_hbm, rels_hbm, tails_hbm, ent_hbm, rel_hbm, out_hbm,
               hidx, ridx, tidx, hpix, rpix, tpix,
               hb0, hb1, rb0, rb1, tb0, tb1, outv, sem0, sem1):
    wid = lax.axis_index("s") * NC + lax.axis_index("c")
    base = wid * BPW

    pltpu.sync_copy(heads_hbm.at[pl.ds(base, BPW)], hidx)
    pltpu.sync_copy(rels_hbm.at[pl.ds(base, BPW)], ridx)
    pltpu.sync_copy(tails_hbm.at[pl.ds(base, BPW)], tidx)

    def fill_pix(v, carry):
        sl = pl.ds(v * L, L)
        c = v // NG
        j = (v % NG) * L
        hpix[c, pl.ds(j, L)] = hidx[sl] // 2
        rpix[c, pl.ds(j, L)] = ridx[sl] // 2
        tpix[c, pl.ds(j, L)] = tidx[sl] // 2
        return carry

    for v in range(BPW // L):
        fill_pix(v, 0)

    hbufs = (hb0, hb1)
    rbufs = (rb0, rb1)
    tbufs = (tb0, tb1)
    sems = (sem0, sem1)

    def issue_chunk(c):
        p = c % 2
        return (
            pltpu.async_copy(ent_hbm.at[hpix.at[c]], hbufs[p], sems[p]),
            pltpu.async_copy(rel_hbm.at[rpix.at[c]], rbufs[p], sems[p]),
            pltpu.async_copy(ent_hbm.at[tpix.at[c]], tbufs[p], sems[p]),
        )

    pending = issue_chunk(0)
    lanes = lax.iota(jnp.int32, L)

    for c in range(NCH):
        for cp in pending:
            cp.wait()
        pending = issue_chunk(c + 1) if c + 1 < NCH else ()
        p = c % 2
        hbuf, rbuf, tbuf = hbufs[p], rbufs[p], tbufs[p]

        def group_body(g, carry, c=c, hbuf=hbuf, rbuf=rbuf, tbuf=tbuf):
            sl = pl.ds(c * CH + g * L, L)
            hpar = (hidx[sl] % 2) * D
            rpar = (ridx[sl] % 2) * D
            tpar = (tidx[sl] % 2) * D
            rows = g * L + lanes
            acc = jnp.zeros((L,), jnp.float32)
            for d in range(D):
                h = plsc.load_gather(hbuf, [rows, hpar + d])
                r = plsc.load_gather(rbuf, [rows, rpar + d])
                t = plsc.load_gather(tbuf, [rows, tpar + d])
                acc = acc + jnp.abs(h + r - t)
            outv[pl.ds(c * CH + g * L, L)] = acc
            return carry

        lax.fori_loop(0, NG, group_body, 0)

    pltpu.sync_copy(outv, out_hbm.at[pl.ds(base, BPW)])


def kernel(heads, relations, tails, entity_table, relation_table):
    ent_pairs = jnp.concatenate(
        [entity_table[0::2], entity_table[1::2]], axis=1)
    rel_pairs = jnp.concatenate(
        [relation_table[0::2], relation_table[1::2]], axis=1)
    return _transe_sc(heads, relations, tails, ent_pairs, rel_pairs)


# speedup vs baseline: 26.5603x; 26.5603x over previous
import functools

import jax
import jax.numpy as jnp
from jax import lax
from jax.experimental import pallas as pl
from jax.experimental.pallas import tpu as pltpu
from jax.experimental.pallas import tpu_sc as plsc

B = 16384
D = 64
L = 16
BAND = 8
P = 2 * D

_info = plsc.get_sparse_core_info()
NC = _info.num_cores
NS = _info.num_subcores
NW = NC * NS
BPW = B // NW
CH = 32
NCH = BPW // CH
NG = CH // L

_mesh = plsc.VectorSubcoreMesh(core_axis_name="c", subcore_axis_name="s")


@functools.partial(
    pl.kernel,
    mesh=_mesh,
    out_type=jax.ShapeDtypeStruct((B,), jnp.float32),
    compiler_params=pltpu.CompilerParams(
        needs_layout_passes=False, use_tc_tiling_on_sc=True),
    scratch_types=[
        pltpu.VMEM((BPW,), jnp.int32),
        pltpu.VMEM((BPW,), jnp.int32),
        pltpu.VMEM((BPW,), jnp.int32),
        pltpu.VMEM((NCH, CH), jnp.int32),
        pltpu.VMEM((CH, BAND, D), jnp.float32),
        pltpu.VMEM((CH, BAND, D), jnp.float32),
        pltpu.VMEM((CH, P), jnp.float32),
        pltpu.VMEM((BPW,), jnp.float32),
        pltpu.SemaphoreType.DMA,
    ],
)
def _transe_sc(heads_hbm, rels_hbm, tails_hbm, ent_hbm, rel_hbm, out_hbm,
               hidx, ridx, tidx, rpix, hb, tb, rb, outv, sem):
    wid = lax.axis_index("s") * NC + lax.axis_index("c")
    base = wid * BPW

    pltpu.sync_copy(heads_hbm.at[pl.ds(base, BPW)], hidx)
    pltpu.sync_copy(rels_hbm.at[pl.ds(base, BPW)], ridx)
    pltpu.sync_copy(tails_hbm.at[pl.ds(base, BPW)], tidx)

    for v in range(BPW // L):
        c = v // NG
        j = (v % NG) * L
        rpix[c, pl.ds(j, L)] = ridx[pl.ds(v * L, L)] // 2

    lanes = lax.iota(jnp.int32, L)

    def chunk_body(c, carry):
        pltpu.async_copy(rel_hbm.at[rpix.at[c]], rb, sem)
        for g in range(NG):
            hvec = hidx[pl.ds(c * CH + g * L, L)] // BAND
            tvec = tidx[pl.ds(c * CH + g * L, L)] // BAND
            for k in range(L):
                j = g * L + k
                pltpu.async_copy(ent_hbm.at[hvec[k]], hb.at[j], sem)
                pltpu.async_copy(ent_hbm.at[tvec[k]], tb.at[j], sem)
        pltpu.make_async_copy(ent_hbm.at[pl.ds(0, CH)], hb, sem).wait()
        pltpu.make_async_copy(ent_hbm.at[pl.ds(0, CH)], tb, sem).wait()
        pltpu.make_async_copy(rel_hbm.at[pl.ds(0, CH)], rb, sem).wait()

        for g in range(NG):
            sl = pl.ds(c * CH + g * L, L)
            hsub = hidx[sl] % BAND
            tsub = tidx[sl] % BAND
            rpar = (ridx[sl] % 2) * D
            rows = g * L + lanes
            acc = jnp.zeros((L,), jnp.float32)
            for d in range(D):
                dvec = jnp.full((L,), d, jnp.int32)
                h = plsc.load_gather(hb, [rows, hsub, dvec])
                t = plsc.load_gather(tb, [rows, tsub, dvec])
                r = plsc.load_gather(rb, [rows, rpar + d])
                acc = acc + jnp.abs(h + r - t)
            outv[pl.ds(c * CH + g * L, L)] = acc
        return carry

    lax.fori_loop(0, NCH, chunk_body, 0)

    pltpu.sync_copy(outv, out_hbm.at[pl.ds(base, BPW)])


def kernel(heads, relations, tails, entity_table, relation_table):
    ent_bands = entity_table.reshape(entity_table.shape[0] // BAND, BAND, D)
    rel_pairs = relation_table.reshape(relation_table.shape[0] // 2, P)
    return _transe_sc(heads, relations, tails, ent_bands, rel_pairs)


# speedup vs baseline: 29.8961x; 1.1256x over previous
import functools

import jax
import jax.numpy as jnp
from jax import lax
from jax.experimental import pallas as pl
from jax.experimental.pallas import tpu as pltpu
from jax.experimental.pallas import tpu_sc as plsc

B = 16384
D = 64
L = 16
BAND = 8
P = 2 * D

_info = plsc.get_sparse_core_info()
NC = _info.num_cores
NS = _info.num_subcores
NW = NC * NS
BPW = B // NW
CH = 16
NCH = BPW // CH
NG = CH // L

_mesh = plsc.VectorSubcoreMesh(core_axis_name="c", subcore_axis_name="s")


@functools.partial(
    pl.kernel,
    mesh=_mesh,
    out_type=jax.ShapeDtypeStruct((B,), jnp.float32),
    compiler_params=pltpu.CompilerParams(
        needs_layout_passes=False, use_tc_tiling_on_sc=True),
    scratch_types=[
        pltpu.VMEM((BPW,), jnp.int32),
        pltpu.VMEM((BPW,), jnp.int32),
        pltpu.VMEM((BPW,), jnp.int32),
        pltpu.VMEM((NCH, CH), jnp.int32),
        pltpu.VMEM((CH, BAND, D), jnp.float32),
        pltpu.VMEM((CH, BAND, D), jnp.float32),
        pltpu.VMEM((CH, BAND, D), jnp.float32),
        pltpu.VMEM((CH, BAND, D), jnp.float32),
        pltpu.VMEM((CH, P), jnp.float32),
        pltpu.VMEM((CH, P), jnp.float32),
        pltpu.VMEM((BPW,), jnp.float32),
        pltpu.SemaphoreType.DMA,
        pltpu.SemaphoreType.DMA,
    ],
)
def _transe_sc(heads_hbm, rels_hbm, tails_hbm, ent_hbm, rel_hbm, out_hbm,
               hidx, ridx, tidx, rpix, hbA, hbB, tbA, tbB, rbA, rbB,
               outv, semA, semB):
    wid = lax.axis_index("s") * NC + lax.axis_index("c")
    base = wid * BPW

    pltpu.sync_copy(heads_hbm.at[pl.ds(base, BPW)], hidx)
    pltpu.sync_copy(rels_hbm.at[pl.ds(base, BPW)], ridx)
    pltpu.sync_copy(tails_hbm.at[pl.ds(base, BPW)], tidx)

    for v in range(BPW // L):
        c = v // NG
        j = (v % NG) * L
        rpix[c, pl.ds(j, L)] = ridx[pl.ds(v * L, L)] // 2

    lanes = lax.iota(jnp.int32, L)

    def issue(c, hb, tb, rb, sem):
        pltpu.async_copy(rel_hbm.at[rpix.at[c]], rb, sem)
        for g in range(NG):
            hvec = hidx[pl.ds(c * CH + g * L, L)] // BAND
            tvec = tidx[pl.ds(c * CH + g * L, L)] // BAND
            for k in range(L):
                j = g * L + k
                pltpu.async_copy(ent_hbm.at[hvec[k]], hb.at[j], sem)
                pltpu.async_copy(ent_hbm.at[tvec[k]], tb.at[j], sem)

    def drain(hb, tb, rb, sem):
        pltpu.make_async_copy(ent_hbm.at[pl.ds(0, CH)], hb, sem).wait()
        pltpu.make_async_copy(ent_hbm.at[pl.ds(0, CH)], tb, sem).wait()
        pltpu.make_async_copy(rel_hbm.at[pl.ds(0, CH)], rb, sem).wait()

    def compute(c, hb, tb, rb):
        for g in range(NG):
            sl = pl.ds(c * CH + g * L, L)
            hsub = hidx[sl] % BAND
            tsub = tidx[sl] % BAND
            rpar = (ridx[sl] % 2) * D
            rows = g * L + lanes
            acc = jnp.zeros((L,), jnp.float32)
            for d in range(D):
                dvec = jnp.full((L,), d, jnp.int32)
                h = plsc.load_gather(hb, [rows, hsub, dvec])
                t = plsc.load_gather(tb, [rows, tsub, dvec])
                r = plsc.load_gather(rb, [rows, rpar + d])
                acc = acc + jnp.abs(h + r - t)
            outv[pl.ds(c * CH + g * L, L)] = acc

    issue(0, hbA, tbA, rbA, semA)

    def pair_body(i, carry):
        issue(2 * i + 1, hbB, tbB, rbB, semB)
        drain(hbA, tbA, rbA, semA)
        compute(2 * i, hbA, tbA, rbA)

        @pl.when(i < NCH // 2 - 1)
        def _():
            issue(2 * i + 2, hbA, tbA, rbA, semA)

        drain(hbB, tbB, rbB, semB)
        compute(2 * i + 1, hbB, tbB, rbB)
        return carry

    lax.fori_loop(0, NCH // 2, pair_body, 0)

    pltpu.sync_copy(outv, out_hbm.at[pl.ds(base, BPW)])


def kernel(heads, relations, tails, entity_table, relation_table):
    ent_bands = entity_table.reshape(entity_table.shape[0] // BAND, BAND, D)
    rel_pairs = relation_table.reshape(relation_table.shape[0] // 2, P)
    return _transe_sc(heads, relations, tails, ent_bands, rel_pairs)
